# Initial kernel scaffold; baseline (speedup 1.0000x reference)
import jax
import jax.numpy as jnp
from jax.experimental import pallas as pl


def kernel(x, edge_index, edge_attr, batch, Wf, bf, Ws, bs, Wlin, blin):
    raise NotImplementedError("write your pallas kernel here")


# speedup vs baseline: 2.0906x; 2.0906x over previous
import functools

import jax
import jax.numpy as jnp
from jax import lax
from jax.experimental import pallas as pl
from jax.experimental.pallas import tpu as pltpu
from jax.experimental.pallas import tpu_sc as plsc

F32 = jnp.float32

N_NODES = 10000
N_EDGES = 320000
D_FEAT = 128
N_GRAPHS = 64
LAYERS = 3

NC = 2
NS = 16
NW = NC * NS
EPW = N_EDGES // NW
CH = 80
NCHUNK = EPW // CH
SLAB = 624
LAST_SLAB = N_NODES - (NS - 1) * SLAB

@functools.cache
def _sc_mesh():
    return plsc.VectorSubcoreMesh(core_axis_name="c", subcore_axis_name="s",
                                  num_cores=NC, num_subcores=NS)

NB = 1000
NGRID = N_NODES // NB
EB = 3200
EGRID = N_EDGES // EB


def _elu(u):
    return jnp.where(u > 0, u, jnp.exp(jnp.minimum(u, 0.0)) - 1.0)


def _proj0_body(h_ref, w_ref, tdst_ref, tsrc_ref, nr_ref):
    t = jnp.dot(h_ref[...], w_ref[...], preferred_element_type=F32)
    tdst_ref[...] = t[:, :256]
    tsrc_ref[...] = t[:, 256:512]
    nr_ref[...] = t[:, 512:]


def _proj_update_body(h_ref, agg_ref, nrin_ref, w_ref,
                      hout_ref, tdst_ref, tsrc_ref, nr_ref):
    h = _elu(h_ref[...] + agg_ref[0] + agg_ref[1])
    hout_ref[...] = h
    t = jnp.dot(h, w_ref[...], preferred_element_type=F32)
    tdst_ref[...] = t[:, :256]
    tsrc_ref[...] = t[:, 256:512]
    nr_ref[...] = nrin_ref[...] + t[:, 512:]


def _act_body(u_ref, ea_ref, we_ref, msg_ref):
    e = jnp.dot(ea_ref[...], we_ref[...], preferred_element_type=F32)
    g = u_ref[:, :128] + e[:, :128]
    s = u_ref[:, 128:] + e[:, 128:]
    gate = 1.0 / (1.0 + jnp.exp(-g))
    sp = jnp.maximum(s, 0.0) + jnp.log(1.0 + jnp.exp(-jnp.abs(s)))
    msg_ref[...] = gate * sp


def _final_body(h_ref, agg_ref, nrin_ref, w_ref, b_ref, batch_ref,
                nr_ref, out_ref):
    i = pl.program_id(0)
    h = _elu(h_ref[...] + agg_ref[0] + agg_ref[1])
    nrb = (nrin_ref[...] + jnp.dot(h, w_ref[...], preferred_element_type=F32)
           + b_ref[0:1, :])
    nr_ref[...] = nrb
    bvec = batch_ref[0, 0, :]
    oh = (lax.broadcasted_iota(jnp.int32, (N_GRAPHS, NB), 0)
          == bvec[None, :]).astype(F32)

    @pl.when(i == 0)
    def _():
        out_ref[...] = jnp.zeros_like(out_ref)

    out_ref[...] += jnp.dot(oh, nrb, preferred_element_type=F32)


def _tc_proj0(h, wcat):
    return pl.pallas_call(
        _proj0_body,
        grid=(NGRID,),
        in_specs=[
            pl.BlockSpec((NB, D_FEAT), lambda i: (i, 0)),
            pl.BlockSpec((D_FEAT, 640), lambda i: (0, 0)),
        ],
        out_specs=[
            pl.BlockSpec((NB, 256), lambda i: (i, 0)),
            pl.BlockSpec((NB, 256), lambda i: (i, 0)),
            pl.BlockSpec((NB, D_FEAT), lambda i: (i, 0)),
        ],
        out_shape=[
            jax.ShapeDtypeStruct((N_NODES, 256), F32),
            jax.ShapeDtypeStruct((N_NODES, 256), F32),
            jax.ShapeDtypeStruct((N_NODES, D_FEAT), F32),
        ],
    )(h, wcat)


def _tc_proj_update(h, agg, nrin, wcat):
    return pl.pallas_call(
        _proj_update_body,
        grid=(NGRID,),
        in_specs=[
            pl.BlockSpec((NB, D_FEAT), lambda i: (i, 0)),
            pl.BlockSpec((NC, NB, D_FEAT), lambda i: (0, i, 0)),
            pl.BlockSpec((NB, D_FEAT), lambda i: (i, 0)),
            pl.BlockSpec((D_FEAT, 640), lambda i: (0, 0)),
        ],
        out_specs=[
            pl.BlockSpec((NB, D_FEAT), lambda i: (i, 0)),
            pl.BlockSpec((NB, 256), lambda i: (i, 0)),
            pl.BlockSpec((NB, 256), lambda i: (i, 0)),
            pl.BlockSpec((NB, D_FEAT), lambda i: (i, 0)),
        ],
        out_shape=[
            jax.ShapeDtypeStruct((N_NODES, D_FEAT), F32),
            jax.ShapeDtypeStruct((N_NODES, 256), F32),
            jax.ShapeDtypeStruct((N_NODES, 256), F32),
            jax.ShapeDtypeStruct((N_NODES, D_FEAT), F32),
        ],
    )(h, agg, nrin, wcat)


def _tc_act(u, ea_aug, we_aug):
    return pl.pallas_call(
        _act_body,
        grid=(EGRID,),
        in_specs=[
            pl.BlockSpec((EB, 256), lambda i: (i, 0)),
            pl.BlockSpec((EB, 17), lambda i: (i, 0)),
            pl.BlockSpec((17, 256), lambda i: (0, 0)),
        ],
        out_specs=pl.BlockSpec((EB, D_FEAT), lambda i: (i, 0)),
        out_shape=jax.ShapeDtypeStruct((N_EDGES, D_FEAT), F32),
    )(u, ea_aug, we_aug)


def _tc_final(h, agg, nrin, wlin3, blin8, batch3):
    return pl.pallas_call(
        _final_body,
        grid=(NGRID,),
        in_specs=[
            pl.BlockSpec((NB, D_FEAT), lambda i: (i, 0)),
            pl.BlockSpec((NC, NB, D_FEAT), lambda i: (0, i, 0)),
            pl.BlockSpec((NB, D_FEAT), lambda i: (i, 0)),
            pl.BlockSpec((D_FEAT, D_FEAT), lambda i: (0, 0)),
            pl.BlockSpec((8, D_FEAT), lambda i: (0, 0)),
            pl.BlockSpec((1, 1, NB), lambda i: (i, 0, 0)),
        ],
        out_specs=[
            pl.BlockSpec((NB, D_FEAT), lambda i: (i, 0)),
            pl.BlockSpec((N_GRAPHS, D_FEAT), lambda i: (0, 0)),
        ],
        out_shape=[
            jax.ShapeDtypeStruct((N_NODES, D_FEAT), F32),
            jax.ShapeDtypeStruct((N_GRAPHS, D_FEAT), F32),
        ],
    )(h, agg, nrin, wlin3, blin8, batch3)


def _sc_gather(tdst, tsrc, dsti, srci):
    @functools.partial(
        pl.kernel,
        out_type=jax.ShapeDtypeStruct((N_EDGES, 256), F32),
        mesh=_sc_mesh(),
        scratch_types=[
            pltpu.VMEM((CH,), jnp.int32),
            pltpu.VMEM((CH,), jnp.int32),
            pltpu.VMEM((CH, 256), F32),
            pltpu.VMEM((CH, 256), F32),
            pltpu.SemaphoreType.DMA,
            pltpu.SemaphoreType.DMA,
        ],
    )
    def k(tdst_hbm, tsrc_hbm, dst_hbm, src_hbm, u_hbm,
          idxd, idxs, bufa, bufb, sema, semb):
        wid = lax.axis_index("s") * NC + lax.axis_index("c")
        base = wid * EPW

        def chunk(i, carry):
            off = base + i * CH
            pltpu.sync_copy(dst_hbm.at[pl.ds(off, CH)], idxd)
            pltpu.sync_copy(src_hbm.at[pl.ds(off, CH)], idxs)
            cpa = pltpu.async_copy(tdst_hbm.at[idxd], bufa, sema)
            cpb = pltpu.async_copy(tsrc_hbm.at[idxs], bufb, semb)
            cpa.wait()
            cpb.wait()

            def row(r, c2):
                for j in range(16):
                    sl = pl.ds(j * 16, 16)
                    bufa[r, sl] = bufa[r, sl] + bufb[r, sl]
                return c2

            lax.fori_loop(0, CH, row, 0)
            pltpu.sync_copy(bufa, u_hbm.at[pl.ds(off, CH)])
            return carry

        lax.fori_loop(0, NCHUNK, chunk, 0)

    return k(tdst, tsrc, dsti, srci)


def _sc_scatter(msg, dsti):
    @functools.partial(
        pl.kernel,
        out_type=jax.ShapeDtypeStruct((NC, N_NODES, D_FEAT), F32),
        mesh=_sc_mesh(),
        scratch_types=[
            pltpu.VMEM((CH,), jnp.int32),
            pltpu.VMEM((CH, D_FEAT), F32),
            pltpu.VMEM((CH, D_FEAT), F32),
            pltpu.VMEM_SHARED((N_NODES, D_FEAT), F32),
        ],
    )
    def k(msg_hbm, dst_hbm, out_hbm, idx, mbuf, zbuf, acc):
        c = lax.axis_index("c")
        s = lax.axis_index("s")
        wid = s * NC + c
        zero = jnp.zeros((16,), F32)

        def zrow(r, cc):
            for j in range(D_FEAT // 16):
                zbuf[r, pl.ds(j * 16, 16)] = zero
            return cc

        lax.fori_loop(0, CH, zrow, 0)

        @pl.when(s < NS - 1)
        def _():
            for t in range(SLAB // CH):
                pltpu.sync_copy(zbuf, acc.at[pl.ds(s * SLAB + t * CH, CH)])
            pltpu.sync_copy(zbuf.at[pl.ds(0, SLAB % CH)],
                            acc.at[pl.ds(s * SLAB + (SLAB // CH) * CH,
                                         SLAB % CH)])

        @pl.when(s == NS - 1)
        def _():
            for t in range(LAST_SLAB // CH):
                pltpu.sync_copy(
                    zbuf, acc.at[pl.ds((NS - 1) * SLAB + t * CH, CH)])

        plsc.subcore_barrier()

        def chunk(i, cc):
            off = wid * EPW + i * CH
            pltpu.sync_copy(dst_hbm.at[pl.ds(off, CH)], idx)
            pltpu.sync_copy(msg_hbm.at[pl.ds(off, CH)], mbuf)
            pltpu.sync_copy(mbuf, acc.at[idx], add=True)
            return cc

        lax.fori_loop(0, NCHUNK, chunk, 0)
        plsc.subcore_barrier()

        @pl.when(s < NS - 1)
        def _():
            pltpu.sync_copy(acc.at[pl.ds(s * SLAB, SLAB)],
                            out_hbm.at[c, pl.ds(s * SLAB, SLAB)])

        @pl.when(s == NS - 1)
        def _():
            pltpu.sync_copy(acc.at[pl.ds((NS - 1) * SLAB, LAST_SLAB)],
                            out_hbm.at[c, pl.ds((NS - 1) * SLAB, LAST_SLAB)])

    return k(msg, dsti)


def kernel(x, edge_index, edge_attr, batch, Wf, bf, Ws, bs, Wlin, blin):
    src = edge_index[0]
    dst = edge_index[1]
    ea_aug = jnp.concatenate(
        [edge_attr, jnp.ones((N_EDGES, 1), F32)], axis=1)
    batch3 = batch.reshape(NGRID, 1, NB)
    blin8 = jnp.broadcast_to(blin[None, :], (8, D_FEAT))

    h = x
    nr = None
    agg = None
    for l in range(LAYERS):
        wcat = jnp.concatenate(
            [Wf[l][:128], Ws[l][:128], Wf[l][128:256], Ws[l][128:256],
             Wlin[128 * l:128 * (l + 1)]], axis=1)
        we_aug = jnp.concatenate(
            [jnp.concatenate([Wf[l][256:], Ws[l][256:]], axis=1),
             jnp.concatenate([bf[l], bs[l]])[None, :]], axis=0)
        if l == 0:
            tdst, tsrc, nr = _tc_proj0(h, wcat)
        else:
            h, tdst, tsrc, nr = _tc_proj_update(h, agg, nr, wcat)
        u = _sc_gather(tdst, tsrc, dst, src)
        msg = _tc_act(u, ea_aug, we_aug)
        agg = _sc_scatter(msg, dst)

    nr_out, out = _tc_final(h, agg, nr, Wlin[384:], blin8, batch3)
    return (out, nr_out)


# speedup vs baseline: 3.3941x; 1.6235x over previous
import functools

import jax
import jax.numpy as jnp
from jax import lax
from jax.experimental import pallas as pl
from jax.experimental.pallas import tpu as pltpu
from jax.experimental.pallas import tpu_sc as plsc

F32 = jnp.float32

N_NODES = 10000
N_EDGES = 320000
D_FEAT = 128
N_GRAPHS = 64
LAYERS = 3

NC = 2
NS = 16
NW = NC * NS
EPW = N_EDGES // NW
CH = 80
NCHUNK = EPW // CH
SLAB = 624
LAST_SLAB = N_NODES - (NS - 1) * SLAB
ZR = 64

@functools.cache
def _sc_mesh():
    return plsc.VectorSubcoreMesh(core_axis_name="c", subcore_axis_name="s",
                                  num_cores=NC, num_subcores=NS)

NB = 1000
NGRID = N_NODES // NB
EB = 3200
EGRID = N_EDGES // EB


def _elu(u):
    return jnp.where(u > 0, u, jnp.exp(jnp.minimum(u, 0.0)) - 1.0)


def _proj0_body(h_ref, w_ref, tdst_ref, tsrc_ref, nr_ref):
    t = jnp.dot(h_ref[...], w_ref[...], preferred_element_type=F32)
    tdst_ref[...] = t[:, :256]
    tsrc_ref[...] = t[:, 256:512]
    nr_ref[...] = t[:, 512:]


def _proj_update_body(h_ref, agg_ref, nrin_ref, w_ref,
                      hout_ref, tdst_ref, tsrc_ref, nr_ref):
    h = _elu(h_ref[...] + agg_ref[0] + agg_ref[1])
    hout_ref[...] = h
    t = jnp.dot(h, w_ref[...], preferred_element_type=F32)
    tdst_ref[...] = t[:, :256]
    tsrc_ref[...] = t[:, 256:512]
    nr_ref[...] = nrin_ref[...] + t[:, 512:]


def _act_body(u_ref, ea_ref, we_ref, msg_ref):
    e = jnp.dot(ea_ref[...], we_ref[...], preferred_element_type=F32)
    g = u_ref[:, :128] + e[:, :128]
    s = u_ref[:, 128:] + e[:, 128:]
    gate = 1.0 / (1.0 + jnp.exp(-g))
    sp = jnp.maximum(s, 0.0) + jnp.log(1.0 + jnp.exp(-jnp.abs(s)))
    msg_ref[...] = gate * sp


def _final_body(h_ref, agg_ref, nrin_ref, w_ref, b_ref, batch_ref,
                nr_ref, out_ref):
    i = pl.program_id(0)
    h = _elu(h_ref[...] + agg_ref[0] + agg_ref[1])
    nrb = (nrin_ref[...] + jnp.dot(h, w_ref[...], preferred_element_type=F32)
           + b_ref[0:1, :])
    nr_ref[...] = nrb
    bvec = batch_ref[0, 0, :]
    oh = (lax.broadcasted_iota(jnp.int32, (N_GRAPHS, NB), 0)
          == bvec[None, :]).astype(F32)

    @pl.when(i == 0)
    def _():
        out_ref[...] = jnp.zeros_like(out_ref)

    out_ref[...] += jnp.dot(oh, nrb, preferred_element_type=F32)


def _tc_proj0(h, wcat):
    return pl.pallas_call(
        _proj0_body,
        grid=(NGRID,),
        in_specs=[
            pl.BlockSpec((NB, D_FEAT), lambda i: (i, 0)),
            pl.BlockSpec((D_FEAT, 640), lambda i: (0, 0)),
        ],
        out_specs=[
            pl.BlockSpec((NB, 256), lambda i: (i, 0)),
            pl.BlockSpec((NB, 256), lambda i: (i, 0)),
            pl.BlockSpec((NB, D_FEAT), lambda i: (i, 0)),
        ],
        out_shape=[
            jax.ShapeDtypeStruct((N_NODES, 256), F32),
            jax.ShapeDtypeStruct((N_NODES, 256), F32),
            jax.ShapeDtypeStruct((N_NODES, D_FEAT), F32),
        ],
    )(h, wcat)


def _tc_proj_update(h, agg, nrin, wcat):
    return pl.pallas_call(
        _proj_update_body,
        grid=(NGRID,),
        in_specs=[
            pl.BlockSpec((NB, D_FEAT), lambda i: (i, 0)),
            pl.BlockSpec((NC, NB, D_FEAT), lambda i: (0, i, 0)),
            pl.BlockSpec((NB, D_FEAT), lambda i: (i, 0)),
            pl.BlockSpec((D_FEAT, 640), lambda i: (0, 0)),
        ],
        out_specs=[
            pl.BlockSpec((NB, D_FEAT), lambda i: (i, 0)),
            pl.BlockSpec((NB, 256), lambda i: (i, 0)),
            pl.BlockSpec((NB, 256), lambda i: (i, 0)),
            pl.BlockSpec((NB, D_FEAT), lambda i: (i, 0)),
        ],
        out_shape=[
            jax.ShapeDtypeStruct((N_NODES, D_FEAT), F32),
            jax.ShapeDtypeStruct((N_NODES, 256), F32),
            jax.ShapeDtypeStruct((N_NODES, 256), F32),
            jax.ShapeDtypeStruct((N_NODES, D_FEAT), F32),
        ],
    )(h, agg, nrin, wcat)


def _tc_act(u, ea_aug, we_aug):
    return pl.pallas_call(
        _act_body,
        grid=(EGRID,),
        in_specs=[
            pl.BlockSpec((EB, 256), lambda i: (i, 0)),
            pl.BlockSpec((EB, 17), lambda i: (i, 0)),
            pl.BlockSpec((17, 256), lambda i: (0, 0)),
        ],
        out_specs=pl.BlockSpec((EB, D_FEAT), lambda i: (i, 0)),
        out_shape=jax.ShapeDtypeStruct((N_EDGES, D_FEAT), F32),
    )(u, ea_aug, we_aug)


def _tc_final(h, agg, nrin, wlin3, blin8, batch3):
    return pl.pallas_call(
        _final_body,
        grid=(NGRID,),
        in_specs=[
            pl.BlockSpec((NB, D_FEAT), lambda i: (i, 0)),
            pl.BlockSpec((NC, NB, D_FEAT), lambda i: (0, i, 0)),
            pl.BlockSpec((NB, D_FEAT), lambda i: (i, 0)),
            pl.BlockSpec((D_FEAT, D_FEAT), lambda i: (0, 0)),
            pl.BlockSpec((8, D_FEAT), lambda i: (0, 0)),
            pl.BlockSpec((1, 1, NB), lambda i: (i, 0, 0)),
        ],
        out_specs=[
            pl.BlockSpec((NB, D_FEAT), lambda i: (i, 0)),
            pl.BlockSpec((N_GRAPHS, D_FEAT), lambda i: (0, 0)),
        ],
        out_shape=[
            jax.ShapeDtypeStruct((N_NODES, D_FEAT), F32),
            jax.ShapeDtypeStruct((N_GRAPHS, D_FEAT), F32),
        ],
    )(h, agg, nrin, wlin3, blin8, batch3)


GCH = 40
GNCH = EPW // GCH
GH = GNCH // 2


def _sc_gather(tdst, tsrc, dsti, srci):
    @functools.partial(
        pl.kernel,
        out_type=jax.ShapeDtypeStruct((N_EDGES, 256), F32),
        mesh=_sc_mesh(),
        scratch_types=[
            pltpu.VMEM((EPW,), jnp.int32),
            pltpu.VMEM((EPW,), jnp.int32),
            pltpu.VMEM((GCH, 256), F32),
            pltpu.VMEM((GCH, 256), F32),
            pltpu.VMEM((GCH, 256), F32),
            pltpu.VMEM((GCH, 256), F32),
            pltpu.VMEM((GCH, 256), F32),
            pltpu.VMEM((GCH, 256), F32),
            pltpu.SemaphoreType.DMA,
            pltpu.SemaphoreType.DMA,
            pltpu.SemaphoreType.DMA,
            pltpu.SemaphoreType.DMA,
            pltpu.SemaphoreType.DMA,
            pltpu.SemaphoreType.DMA,
        ],
    )
    def k(tdst_hbm, tsrc_hbm, dst_hbm, src_hbm, u_hbm,
          idxd, idxs, a0, a1, b0, b1, st0, st1,
          sa0, sa1, sb0, sb1, ss0, ss1):
        wid = lax.axis_index("s") * NC + lax.axis_index("c")
        base = wid * EPW
        pltpu.sync_copy(dst_hbm.at[pl.ds(base, EPW)], idxd)
        pltpu.sync_copy(src_hbm.at[pl.ds(base, EPW)], idxs)

        slots = ((a0, b0, st0, sa0, sb0, ss0),
                 (a1, b1, st1, sa1, sb1, ss1))

        def fire(i, sl):
            a, b, _, sa, sb, _ = sl
            pltpu.async_copy(
                tdst_hbm.at[idxd.at[pl.ds(i * GCH, GCH)]], a, sa)
            pltpu.async_copy(
                tsrc_hbm.at[idxs.at[pl.ds(i * GCH, GCH)]], b, sb)

        fire(0, slots[0])
        fire(1, slots[1])

        def handle(g, i, sl):
            a, b, st, sa, sb, ss = sl
            off = base + i * GCH
            pltpu.make_async_copy(
                tdst_hbm.at[idxd.at[pl.ds(0, GCH)]], a, sa).wait()
            pltpu.make_async_copy(
                tsrc_hbm.at[idxs.at[pl.ds(0, GCH)]], b, sb).wait()

            @pl.when(g >= 1)
            def _():
                pltpu.make_async_copy(st, u_hbm.at[pl.ds(0, GCH)], ss).wait()

            def row(r, cc):
                for j in range(16):
                    c16 = pl.ds(j * 16, 16)
                    st[r, c16] = a[r, c16] + b[r, c16]
                return cc

            lax.fori_loop(0, GCH, row, 0)
            pltpu.async_copy(st, u_hbm.at[pl.ds(off, GCH)], ss)

            @pl.when(i + 2 < GNCH)
            def _():
                fire(i + 2, sl)

        def body(g, cc):
            handle(g, 2 * g, slots[0])
            handle(g, 2 * g + 1, slots[1])
            return cc

        lax.fori_loop(0, GH, body, 0)
        pltpu.make_async_copy(st0, u_hbm.at[pl.ds(0, GCH)], ss0).wait()
        pltpu.make_async_copy(st1, u_hbm.at[pl.ds(0, GCH)], ss1).wait()

    return k(tdst, tsrc, dsti, srci)


def _sc_scatter(msg, dsti):
    @functools.partial(
        pl.kernel,
        out_type=jax.ShapeDtypeStruct((NC, N_NODES, D_FEAT), F32),
        mesh=_sc_mesh(),
        scratch_types=[
            pltpu.VMEM((CH,), jnp.int32),
            pltpu.VMEM((CH,), jnp.int32),
            pltpu.VMEM((CH,), jnp.int32),
            pltpu.VMEM((CH,), jnp.int32),
            pltpu.VMEM((CH, D_FEAT), F32),
            pltpu.VMEM((CH, D_FEAT), F32),
            pltpu.VMEM((CH, D_FEAT), F32),
            pltpu.VMEM((CH, D_FEAT), F32),
            pltpu.VMEM((ZR, D_FEAT), F32),
            pltpu.VMEM_SHARED((N_NODES, D_FEAT), F32),
            pltpu.SemaphoreType.DMA,
            pltpu.SemaphoreType.DMA,
            pltpu.SemaphoreType.DMA,
            pltpu.SemaphoreType.DMA,
            pltpu.SemaphoreType.DMA,
            pltpu.SemaphoreType.DMA,
            pltpu.SemaphoreType.DMA,
            pltpu.SemaphoreType.DMA,
            pltpu.SemaphoreType.DMA,
            pltpu.SemaphoreType.DMA,
            pltpu.SemaphoreType.DMA,
            pltpu.SemaphoreType.DMA,
        ],
    )
    def k(msg_hbm, dst_hbm, out_hbm, i0, i1, i2, i3, m0, m1, m2, m3,
          zbuf, acc, si0, si1, si2, si3, sm0, sm1, sm2, sm3,
          sc0, sc1, sc2, sc3):
        c = lax.axis_index("c")
        s = lax.axis_index("s")
        wid = s * NC + c
        zero = jnp.zeros((16,), F32)
        ibufs = (i0, i1, i2, i3)
        mbufs = (m0, m1, m2, m3)
        sidx = (si0, si1, si2, si3)
        smsg = (sm0, sm1, sm2, sm3)
        ssc = (sc0, sc1, sc2, sc3)

        def zrow(r, cc):
            for j in range(D_FEAT // 16):
                zbuf[r, pl.ds(j * 16, 16)] = zero
            return cc

        lax.fori_loop(0, ZR, zrow, 0)

        @pl.when(s < NS - 1)
        def _():
            for t in range(SLAB // ZR):
                pltpu.async_copy(zbuf, acc.at[pl.ds(s * SLAB + t * ZR, ZR)],
                                 sc0)
            pltpu.async_copy(zbuf.at[pl.ds(0, SLAB % ZR)],
                             acc.at[pl.ds(s * SLAB + (SLAB // ZR) * ZR,
                                          SLAB % ZR)], sc1)
            for t in range(SLAB // ZR):
                pltpu.make_async_copy(
                    zbuf, acc.at[pl.ds(s * SLAB, ZR)], sc0).wait()
            pltpu.make_async_copy(
                zbuf.at[pl.ds(0, SLAB % ZR)],
                acc.at[pl.ds(s * SLAB, SLAB % ZR)], sc1).wait()

        @pl.when(s == NS - 1)
        def _():
            for t in range(LAST_SLAB // ZR):
                pltpu.async_copy(
                    zbuf, acc.at[pl.ds((NS - 1) * SLAB + t * ZR, ZR)], sc0)
            for t in range(LAST_SLAB // ZR):
                pltpu.make_async_copy(
                    zbuf, acc.at[pl.ds((NS - 1) * SLAB, ZR)], sc0).wait()

        plsc.subcore_barrier()

        def fire_load(i, p):
            off = wid * EPW + i * CH
            pltpu.async_copy(dst_hbm.at[pl.ds(off, CH)], ibufs[p], sidx[p])
            pltpu.async_copy(msg_hbm.at[pl.ds(off, CH)], mbufs[p], smsg[p])

        fire_load(0, 0)
        fire_load(1, 1)

        def do_chunk(i, p):
            q = (p + 2) % 4
            pltpu.make_async_copy(
                dst_hbm.at[pl.ds(0, CH)], ibufs[p], sidx[p]).wait()
            pltpu.make_async_copy(
                msg_hbm.at[pl.ds(0, CH)], mbufs[p], smsg[p]).wait()
            pltpu.async_copy(mbufs[p], acc.at[ibufs[p]], ssc[p], add=True)

            @pl.when(i + 2 < NCHUNK)
            def _():
                @pl.when(i >= 2)
                def _():
                    pltpu.make_async_copy(
                        mbufs[q], acc.at[ibufs[q]], ssc[q]).wait()

                fire_load(i + 2, q)

        pltpu.make_async_copy(dst_hbm.at[pl.ds(0, CH)], ibufs[0], sidx[0]).wait()
        pltpu.make_async_copy(msg_hbm.at[pl.ds(0, CH)], mbufs[0], smsg[0]).wait()
        pltpu.async_copy(mbufs[0], acc.at[ibufs[0]], ssc[0], add=True)
        fire_load(2, 2)

        def body(g, cc):
            for t in range(4):
                do_chunk(4 * g + 1 + t, (1 + t) % 4)
            return cc

        lax.fori_loop(0, (NCHUNK - 1) // 4, body, 0)
        for p in range(4):
            pltpu.make_async_copy(
                mbufs[p], acc.at[ibufs[p]], ssc[p]).wait()
        plsc.subcore_barrier()

        @pl.when(s < NS - 1)
        def _():
            pltpu.sync_copy(acc.at[pl.ds(s * SLAB, SLAB)],
                            out_hbm.at[c, pl.ds(s * SLAB, SLAB)])

        @pl.when(s == NS - 1)
        def _():
            pltpu.sync_copy(acc.at[pl.ds((NS - 1) * SLAB, LAST_SLAB)],
                            out_hbm.at[c, pl.ds((NS - 1) * SLAB, LAST_SLAB)])

    return k(msg, dsti)


def kernel(x, edge_index, edge_attr, batch, Wf, bf, Ws, bs, Wlin, blin):
    src = edge_index[0]
    dst = edge_index[1]
    ea_aug = jnp.concatenate(
        [edge_attr, jnp.ones((N_EDGES, 1), F32)], axis=1)
    batch3 = batch.reshape(NGRID, 1, NB)
    blin8 = jnp.broadcast_to(blin[None, :], (8, D_FEAT))

    h = x
    nr = None
    agg = None
    for l in range(LAYERS):
        wcat = jnp.concatenate(
            [Wf[l][:128], Ws[l][:128], Wf[l][128:256], Ws[l][128:256],
             Wlin[128 * l:128 * (l + 1)]], axis=1)
        we_aug = jnp.concatenate(
            [jnp.concatenate([Wf[l][256:], Ws[l][256:]], axis=1),
             jnp.concatenate([bf[l], bs[l]])[None, :]], axis=0)
        if l == 0:
            tdst, tsrc, nr = _tc_proj0(h, wcat)
        else:
            h, tdst, tsrc, nr = _tc_proj_update(h, agg, nr, wcat)
        u = _sc_gather(tdst, tsrc, dst, src)
        msg = _tc_act(u, ea_aug, we_aug)
        agg = _sc_scatter(msg, dst)

    nr_out, out = _tc_final(h, agg, nr, Wlin[384:], blin8, batch3)
    return (out, nr_out)


# speedup vs baseline: 3.8744x; 1.1415x over previous
import functools

import jax
import jax.numpy as jnp
from jax import lax
from jax.experimental import pallas as pl
from jax.experimental.pallas import tpu as pltpu
from jax.experimental.pallas import tpu_sc as plsc

F32 = jnp.float32
BF16 = jnp.bfloat16

N_NODES = 10000
N_EDGES = 320000
D_FEAT = 128
N_GRAPHS = 64
LAYERS = 3

NC = 2
NS = 16
NW = NC * NS
EPW = N_EDGES // NW
CH = 80
NCHUNK = EPW // CH
SLAB = 624
LAST_SLAB = N_NODES - (NS - 1) * SLAB
ZR = 64

@functools.cache
def _sc_mesh():
    return plsc.VectorSubcoreMesh(core_axis_name="c", subcore_axis_name="s",
                                  num_cores=NC, num_subcores=NS)

NB = 1000
NGRID = N_NODES // NB
EB = 3200
EGRID = N_EDGES // EB


def _elu(u):
    return jnp.where(u > 0, u, jnp.exp(jnp.minimum(u, 0.0)) - 1.0)


def _pack2bf16(g, c):
    gu = lax.bitcast_convert_type(g.astype(BF16), jnp.uint16).astype(jnp.uint32)
    cu = lax.bitcast_convert_type(c.astype(BF16), jnp.uint16).astype(jnp.uint32)
    return lax.bitcast_convert_type(gu | (cu << 16), jnp.int32)


def _unpack2bf16(p):
    pu = lax.bitcast_convert_type(p, jnp.uint32)
    g = lax.bitcast_convert_type((pu & 0xFFFF).astype(jnp.uint16), BF16)
    c = lax.bitcast_convert_type((pu >> 16).astype(jnp.uint16), BF16)
    return g.astype(F32), c.astype(F32)


def _proj0_body(h_ref, w_ref, tdst_ref, tsrc_ref, nr_ref):
    t = jnp.dot(h_ref[...], w_ref[...], preferred_element_type=F32)
    tdst_ref[...] = _pack2bf16(t[:, :128], t[:, 128:256])
    tsrc_ref[...] = _pack2bf16(t[:, 256:384], t[:, 384:512])
    nr_ref[...] = t[:, 512:]


def _proj_update_body(h_ref, agg_ref, nrin_ref, w_ref,
                      hout_ref, tdst_ref, tsrc_ref, nr_ref):
    h = _elu(h_ref[...] + agg_ref[0] + agg_ref[1])
    hout_ref[...] = h
    t = jnp.dot(h, w_ref[...], preferred_element_type=F32)
    tdst_ref[...] = _pack2bf16(t[:, :128], t[:, 128:256])
    tsrc_ref[...] = _pack2bf16(t[:, 256:384], t[:, 384:512])
    nr_ref[...] = nrin_ref[...] + t[:, 512:]


def _act_body(u1_ref, u2_ref, ea_ref, we_ref, msg_ref):
    e = jnp.dot(ea_ref[...], we_ref[...], preferred_element_type=F32)
    g1, c1 = _unpack2bf16(u1_ref[...])
    g2, c2 = _unpack2bf16(u2_ref[...])
    g = g1 + g2 + e[:, :128]
    s = c1 + c2 + e[:, 128:]
    gate = 1.0 / (1.0 + jnp.exp(-g))
    sp = jnp.maximum(s, 0.0) + jnp.log(1.0 + jnp.exp(-jnp.abs(s)))
    msg_ref[...] = gate * sp


def _final_body(h_ref, agg_ref, nrin_ref, w_ref, b_ref, batch_ref,
                nr_ref, out_ref):
    i = pl.program_id(0)
    h = _elu(h_ref[...] + agg_ref[0] + agg_ref[1])
    nrb = (nrin_ref[...] + jnp.dot(h, w_ref[...], preferred_element_type=F32)
           + b_ref[0:1, :])
    nr_ref[...] = nrb
    bvec = batch_ref[0, 0, :]
    oh = (lax.broadcasted_iota(jnp.int32, (N_GRAPHS, NB), 0)
          == bvec[None, :]).astype(F32)

    @pl.when(i == 0)
    def _():
        out_ref[...] = jnp.zeros_like(out_ref)

    out_ref[...] += jnp.dot(oh, nrb, preferred_element_type=F32)


def _tc_proj0(h, wcat):
    return pl.pallas_call(
        _proj0_body,
        grid=(NGRID,),
        in_specs=[
            pl.BlockSpec((NB, D_FEAT), lambda i: (i, 0)),
            pl.BlockSpec((D_FEAT, 640), lambda i: (0, 0)),
        ],
        out_specs=[
            pl.BlockSpec((NB, 128), lambda i: (i, 0)),
            pl.BlockSpec((NB, 128), lambda i: (i, 0)),
            pl.BlockSpec((NB, D_FEAT), lambda i: (i, 0)),
        ],
        out_shape=[
            jax.ShapeDtypeStruct((N_NODES, 128), jnp.int32),
            jax.ShapeDtypeStruct((N_NODES, 128), jnp.int32),
            jax.ShapeDtypeStruct((N_NODES, D_FEAT), F32),
        ],
    )(h, wcat)


def _tc_proj_update(h, agg, nrin, wcat):
    return pl.pallas_call(
        _proj_update_body,
        grid=(NGRID,),
        in_specs=[
            pl.BlockSpec((NB, D_FEAT), lambda i: (i, 0)),
            pl.BlockSpec((NC, NB, D_FEAT), lambda i: (0, i, 0)),
            pl.BlockSpec((NB, D_FEAT), lambda i: (i, 0)),
            pl.BlockSpec((D_FEAT, 640), lambda i: (0, 0)),
        ],
        out_specs=[
            pl.BlockSpec((NB, D_FEAT), lambda i: (i, 0)),
            pl.BlockSpec((NB, 128), lambda i: (i, 0)),
            pl.BlockSpec((NB, 128), lambda i: (i, 0)),
            pl.BlockSpec((NB, D_FEAT), lambda i: (i, 0)),
        ],
        out_shape=[
            jax.ShapeDtypeStruct((N_NODES, D_FEAT), F32),
            jax.ShapeDtypeStruct((N_NODES, 128), jnp.int32),
            jax.ShapeDtypeStruct((N_NODES, 128), jnp.int32),
            jax.ShapeDtypeStruct((N_NODES, D_FEAT), F32),
        ],
    )(h, agg, nrin, wcat)


def _tc_act(u1, u2, ea_aug, we_aug):
    return pl.pallas_call(
        _act_body,
        grid=(EGRID,),
        in_specs=[
            pl.BlockSpec((EB, 128), lambda i: (i, 0)),
            pl.BlockSpec((EB, 128), lambda i: (i, 0)),
            pl.BlockSpec((EB, 17), lambda i: (i, 0)),
            pl.BlockSpec((17, 256), lambda i: (0, 0)),
        ],
        out_specs=pl.BlockSpec((EB, D_FEAT), lambda i: (i, 0)),
        out_shape=jax.ShapeDtypeStruct((N_EDGES, D_FEAT), F32),
    )(u1, u2, ea_aug, we_aug)


def _tc_final(h, agg, nrin, wlin3, blin8, batch3):
    return pl.pallas_call(
        _final_body,
        grid=(NGRID,),
        in_specs=[
            pl.BlockSpec((NB, D_FEAT), lambda i: (i, 0)),
            pl.BlockSpec((NC, NB, D_FEAT), lambda i: (0, i, 0)),
            pl.BlockSpec((NB, D_FEAT), lambda i: (i, 0)),
            pl.BlockSpec((D_FEAT, D_FEAT), lambda i: (0, 0)),
            pl.BlockSpec((8, D_FEAT), lambda i: (0, 0)),
            pl.BlockSpec((1, 1, NB), lambda i: (i, 0, 0)),
        ],
        out_specs=[
            pl.BlockSpec((NB, D_FEAT), lambda i: (i, 0)),
            pl.BlockSpec((N_GRAPHS, D_FEAT), lambda i: (0, 0)),
        ],
        out_shape=[
            jax.ShapeDtypeStruct((N_NODES, D_FEAT), F32),
            jax.ShapeDtypeStruct((N_GRAPHS, D_FEAT), F32),
        ],
    )(h, agg, nrin, wlin3, blin8, batch3)


GCH = 40
GNCH = EPW // GCH
GH = GNCH // 2


def _sc_gather(tdst, tsrc, dsti, srci):
    @functools.partial(
        pl.kernel,
        out_type=[jax.ShapeDtypeStruct((N_EDGES, 128), jnp.int32),
                  jax.ShapeDtypeStruct((N_EDGES, 128), jnp.int32)],
        mesh=_sc_mesh(),
        scratch_types=(
            [pltpu.VMEM((EPW,), jnp.int32)] * 2
            + [pltpu.VMEM((GCH, 128), jnp.int32)] * 8
            + [pltpu.SemaphoreType.DMA] * 16
        ),
    )
    def k(tdst_hbm, tsrc_hbm, dst_hbm, src_hbm, u1_hbm, u2_hbm,
          idxd, idxs, a0, a1, a2, a3, b0, b1, b2, b3,
          sa0, sa1, sa2, sa3, sb0, sb1, sb2, sb3,
          ta0, ta1, ta2, ta3, tb0, tb1, tb2, tb3):
        wid = lax.axis_index("s") * NC + lax.axis_index("c")
        base = wid * EPW
        pltpu.sync_copy(dst_hbm.at[pl.ds(base, EPW)], idxd)
        pltpu.sync_copy(src_hbm.at[pl.ds(base, EPW)], idxs)
        abuf = (a0, a1, a2, a3)
        bbuf = (b0, b1, b2, b3)
        sga = (sa0, sa1, sa2, sa3)
        sgb = (sb0, sb1, sb2, sb3)
        sta = (ta0, ta1, ta2, ta3)
        stb = (tb0, tb1, tb2, tb3)

        def fire(i, p):
            pltpu.async_copy(
                tdst_hbm.at[idxd.at[pl.ds(i * GCH, GCH)]], abuf[p], sga[p])
            pltpu.async_copy(
                tsrc_hbm.at[idxs.at[pl.ds(i * GCH, GCH)]], bbuf[p], sgb[p])

        def visit(i, p, first):
            q = (p + 2) % 4
            off = base + i * GCH
            pltpu.make_async_copy(
                tdst_hbm.at[idxd.at[pl.ds(0, GCH)]], abuf[p], sga[p]).wait()
            pltpu.make_async_copy(
                tsrc_hbm.at[idxs.at[pl.ds(0, GCH)]], bbuf[p], sgb[p]).wait()
            pltpu.async_copy(abuf[p], u1_hbm.at[pl.ds(off, GCH)], sta[p])
            pltpu.async_copy(bbuf[p], u2_hbm.at[pl.ds(off, GCH)], stb[p])
            if first:
                fire(i + 2, q)
            else:
                pltpu.make_async_copy(
                    abuf[q], u1_hbm.at[pl.ds(0, GCH)], sta[q]).wait()
                pltpu.make_async_copy(
                    bbuf[q], u2_hbm.at[pl.ds(0, GCH)], stb[q]).wait()

                @pl.when(i + 2 < GNCH)
                def _():
                    fire(i + 2, q)

        fire(0, 0)
        fire(1, 1)
        visit(0, 0, True)
        visit(1, 1, True)

        def body(g, cc):
            for t in range(4):
                visit(4 * g + 2 + t, (2 + t) % 4, False)
            return cc

        lax.fori_loop(0, (GNCH - 2) // 4, body, 0)
        for i in (GNCH - 2, GNCH - 1):
            p = i % 4
            pltpu.make_async_copy(
                abuf[p], u1_hbm.at[pl.ds(0, GCH)], sta[p]).wait()
            pltpu.make_async_copy(
                bbuf[p], u2_hbm.at[pl.ds(0, GCH)], stb[p]).wait()

    return k(tdst, tsrc, dsti, srci)


def _sc_scatter(msg, dsti):
    @functools.partial(
        pl.kernel,
        out_type=jax.ShapeDtypeStruct((NC, N_NODES, D_FEAT), F32),
        mesh=_sc_mesh(),
        scratch_types=[
            pltpu.VMEM((CH,), jnp.int32),
            pltpu.VMEM((CH,), jnp.int32),
            pltpu.VMEM((CH,), jnp.int32),
            pltpu.VMEM((CH,), jnp.int32),
            pltpu.VMEM((CH, D_FEAT), F32),
            pltpu.VMEM((CH, D_FEAT), F32),
            pltpu.VMEM((CH, D_FEAT), F32),
            pltpu.VMEM((CH, D_FEAT), F32),
            pltpu.VMEM((ZR, D_FEAT), F32),
            pltpu.VMEM_SHARED((N_NODES, D_FEAT), F32),
            pltpu.SemaphoreType.DMA,
            pltpu.SemaphoreType.DMA,
            pltpu.SemaphoreType.DMA,
            pltpu.SemaphoreType.DMA,
            pltpu.SemaphoreType.DMA,
            pltpu.SemaphoreType.DMA,
            pltpu.SemaphoreType.DMA,
            pltpu.SemaphoreType.DMA,
            pltpu.SemaphoreType.DMA,
            pltpu.SemaphoreType.DMA,
            pltpu.SemaphoreType.DMA,
            pltpu.SemaphoreType.DMA,
        ],
    )
    def k(msg_hbm, dst_hbm, out_hbm, i0, i1, i2, i3, m0, m1, m2, m3,
          zbuf, acc, si0, si1, si2, si3, sm0, sm1, sm2, sm3,
          sc0, sc1, sc2, sc3):
        c = lax.axis_index("c")
        s = lax.axis_index("s")
        wid = s * NC + c
        zero = jnp.zeros((16,), F32)
        ibufs = (i0, i1, i2, i3)
        mbufs = (m0, m1, m2, m3)
        sidx = (si0, si1, si2, si3)
        smsg = (sm0, sm1, sm2, sm3)
        ssc = (sc0, sc1, sc2, sc3)

        def zrow(r, cc):
            for j in range(D_FEAT // 16):
                zbuf[r, pl.ds(j * 16, 16)] = zero
            return cc

        lax.fori_loop(0, ZR, zrow, 0)

        @pl.when(s < NS - 1)
        def _():
            for t in range(SLAB // ZR):
                pltpu.async_copy(zbuf, acc.at[pl.ds(s * SLAB + t * ZR, ZR)],
                                 sc0)
            pltpu.async_copy(zbuf.at[pl.ds(0, SLAB % ZR)],
                             acc.at[pl.ds(s * SLAB + (SLAB // ZR) * ZR,
                                          SLAB % ZR)], sc1)
            for t in range(SLAB // ZR):
                pltpu.make_async_copy(
                    zbuf, acc.at[pl.ds(s * SLAB, ZR)], sc0).wait()
            pltpu.make_async_copy(
                zbuf.at[pl.ds(0, SLAB % ZR)],
                acc.at[pl.ds(s * SLAB, SLAB % ZR)], sc1).wait()

        @pl.when(s == NS - 1)
        def _():
            for t in range(LAST_SLAB // ZR):
                pltpu.async_copy(
                    zbuf, acc.at[pl.ds((NS - 1) * SLAB + t * ZR, ZR)], sc0)
            for t in range(LAST_SLAB // ZR):
                pltpu.make_async_copy(
                    zbuf, acc.at[pl.ds((NS - 1) * SLAB, ZR)], sc0).wait()

        plsc.subcore_barrier()

        def fire_load(i, p):
            off = wid * EPW + i * CH
            pltpu.async_copy(dst_hbm.at[pl.ds(off, CH)], ibufs[p], sidx[p])
            pltpu.async_copy(msg_hbm.at[pl.ds(off, CH)], mbufs[p], smsg[p])

        fire_load(0, 0)
        fire_load(1, 1)

        def do_chunk(i, p):
            q = (p + 2) % 4
            pltpu.make_async_copy(
                dst_hbm.at[pl.ds(0, CH)], ibufs[p], sidx[p]).wait()
            pltpu.make_async_copy(
                msg_hbm.at[pl.ds(0, CH)], mbufs[p], smsg[p]).wait()
            pltpu.async_copy(mbufs[p], acc.at[ibufs[p]], ssc[p], add=True)

            @pl.when(i + 2 < NCHUNK)
            def _():
                @pl.when(i >= 2)
                def _():
                    pltpu.make_async_copy(
                        mbufs[q], acc.at[ibufs[q]], ssc[q]).wait()

                fire_load(i + 2, q)

        pltpu.make_async_copy(dst_hbm.at[pl.ds(0, CH)], ibufs[0], sidx[0]).wait()
        pltpu.make_async_copy(msg_hbm.at[pl.ds(0, CH)], mbufs[0], smsg[0]).wait()
        pltpu.async_copy(mbufs[0], acc.at[ibufs[0]], ssc[0], add=True)
        fire_load(2, 2)

        def body(g, cc):
            for t in range(4):
                do_chunk(4 * g + 1 + t, (1 + t) % 4)
            return cc

        lax.fori_loop(0, (NCHUNK - 1) // 4, body, 0)
        for p in range(4):
            pltpu.make_async_copy(
                mbufs[p], acc.at[ibufs[p]], ssc[p]).wait()
        plsc.subcore_barrier()

        @pl.when(s < NS - 1)
        def _():
            pltpu.sync_copy(acc.at[pl.ds(s * SLAB, SLAB)],
                            out_hbm.at[c, pl.ds(s * SLAB, SLAB)])

        @pl.when(s == NS - 1)
        def _():
            pltpu.sync_copy(acc.at[pl.ds((NS - 1) * SLAB, LAST_SLAB)],
                            out_hbm.at[c, pl.ds((NS - 1) * SLAB, LAST_SLAB)])

    return k(msg, dsti)


def kernel(x, edge_index, edge_attr, batch, Wf, bf, Ws, bs, Wlin, blin):
    src = edge_index[0]
    dst = edge_index[1]
    ea_aug = jnp.concatenate(
        [edge_attr, jnp.ones((N_EDGES, 1), F32)], axis=1)
    batch3 = batch.reshape(NGRID, 1, NB)
    blin8 = jnp.broadcast_to(blin[None, :], (8, D_FEAT))

    h = x
    nr = None
    agg = None
    for l in range(LAYERS):
        wcat = jnp.concatenate(
            [Wf[l][:128], Ws[l][:128], Wf[l][128:256], Ws[l][128:256],
             Wlin[128 * l:128 * (l + 1)]], axis=1)
        we_aug = jnp.concatenate(
            [jnp.concatenate([Wf[l][256:], Ws[l][256:]], axis=1),
             jnp.concatenate([bf[l], bs[l]])[None, :]], axis=0)
        if l == 0:
            tdst, tsrc, nr = _tc_proj0(h, wcat)
        else:
            h, tdst, tsrc, nr = _tc_proj_update(h, agg, nr, wcat)
        u1, u2 = _sc_gather(tdst, tsrc, dst, src)
        msg = _tc_act(u1, u2, ea_aug, we_aug)
        agg = _sc_scatter(msg, dst)

    nr_out, out = _tc_final(h, agg, nr, Wlin[384:], blin8, batch3)
    return (out, nr_out)


# speedup vs baseline: 4.0805x; 1.0532x over previous
import functools

import jax
import jax.numpy as jnp
from jax import lax
from jax.experimental import pallas as pl
from jax.experimental.pallas import tpu as pltpu
from jax.experimental.pallas import tpu_sc as plsc

F32 = jnp.float32
BF16 = jnp.bfloat16

N_NODES = 10000
N_EDGES = 320000
D_FEAT = 128
N_GRAPHS = 64
LAYERS = 3

NC = 2
NS = 16
NW = NC * NS
EPW = N_EDGES // NW
CH = 80
NCHUNK = EPW // CH
SLAB = 624
LAST_SLAB = N_NODES - (NS - 1) * SLAB
ZR = 64

@functools.cache
def _sc_mesh():
    return plsc.VectorSubcoreMesh(core_axis_name="c", subcore_axis_name="s",
                                  num_cores=NC, num_subcores=NS)

NB = 1000
NGRID = N_NODES // NB
EB = 3200


def _elu(u):
    return jnp.where(u > 0, u, jnp.exp(jnp.minimum(u, 0.0)) - 1.0)


def _pack2bf16(g, c):
    gu = lax.bitcast_convert_type(g.astype(BF16), jnp.uint16).astype(jnp.uint32)
    cu = lax.bitcast_convert_type(c.astype(BF16), jnp.uint16).astype(jnp.uint32)
    return lax.bitcast_convert_type(gu | (cu << 16), jnp.int32)


def _unpack2bf16(p):
    pu = lax.bitcast_convert_type(p, jnp.uint32)
    g = lax.bitcast_convert_type((pu & 0xFFFF).astype(jnp.uint16), BF16)
    c = lax.bitcast_convert_type((pu >> 16).astype(jnp.uint16), BF16)
    return g.astype(F32), c.astype(F32)


def _proj0_body(h_ref, w_ref, tdst_ref, tsrc_ref, nr_ref):
    t = jnp.dot(h_ref[...], w_ref[...], preferred_element_type=F32)
    tdst_ref[...] = _pack2bf16(t[:, :128], t[:, 128:256])
    tsrc_ref[...] = _pack2bf16(t[:, 256:384], t[:, 384:512])
    nr_ref[...] = t[:, 512:]


def _proj_update_body(h_ref, agga_ref, aggb_ref, nrin_ref, w_ref,
                      hout_ref, tdst_ref, tsrc_ref, nr_ref):
    h = _elu(h_ref[...] + agga_ref[0] + agga_ref[1]
             + aggb_ref[0] + aggb_ref[1])
    hout_ref[...] = h
    t = jnp.dot(h, w_ref[...], preferred_element_type=F32)
    tdst_ref[...] = _pack2bf16(t[:, :128], t[:, 128:256])
    tsrc_ref[...] = _pack2bf16(t[:, 256:384], t[:, 384:512])
    nr_ref[...] = nrin_ref[...] + t[:, 512:]


def _act_body(u1_ref, u2_ref, ea_ref, we_ref, msg_ref):
    e = jnp.dot(ea_ref[...], we_ref[...], preferred_element_type=F32)
    g1, c1 = _unpack2bf16(u1_ref[...])
    g2, c2 = _unpack2bf16(u2_ref[...])
    g = g1 + g2 + e[:, :128]
    s = c1 + c2 + e[:, 128:]
    gate = 1.0 / (1.0 + jnp.exp(-g))
    sp = jnp.maximum(s, 0.0) + jnp.log(1.0 + jnp.exp(-jnp.abs(s)))
    msg_ref[...] = gate * sp


def _final_body(h_ref, agga_ref, aggb_ref, nrin_ref, w_ref, b_ref,
                batch_ref, nr_ref, out_ref):
    i = pl.program_id(0)
    h = _elu(h_ref[...] + agga_ref[0] + agga_ref[1]
             + aggb_ref[0] + aggb_ref[1])
    nrb = (nrin_ref[...] + jnp.dot(h, w_ref[...], preferred_element_type=F32)
           + b_ref[0:1, :])
    nr_ref[...] = nrb
    bvec = batch_ref[0, 0, :]
    oh = (lax.broadcasted_iota(jnp.int32, (N_GRAPHS, NB), 0)
          == bvec[None, :]).astype(F32)

    @pl.when(i == 0)
    def _():
        out_ref[...] = jnp.zeros_like(out_ref)

    out_ref[...] += jnp.dot(oh, nrb, preferred_element_type=F32)


def _tc_proj0(h, wcat):
    return pl.pallas_call(
        _proj0_body,
        grid=(NGRID,),
        in_specs=[
            pl.BlockSpec((NB, D_FEAT), lambda i: (i, 0)),
            pl.BlockSpec((D_FEAT, 640), lambda i: (0, 0)),
        ],
        out_specs=[
            pl.BlockSpec((NB, 128), lambda i: (i, 0)),
            pl.BlockSpec((NB, 128), lambda i: (i, 0)),
            pl.BlockSpec((NB, D_FEAT), lambda i: (i, 0)),
        ],
        out_shape=[
            jax.ShapeDtypeStruct((N_NODES, 128), jnp.int32),
            jax.ShapeDtypeStruct((N_NODES, 128), jnp.int32),
            jax.ShapeDtypeStruct((N_NODES, D_FEAT), F32),
        ],
    )(h, wcat)


def _tc_proj_update(h, agga, aggb, nrin, wcat):
    return pl.pallas_call(
        _proj_update_body,
        grid=(NGRID,),
        in_specs=[
            pl.BlockSpec((NB, D_FEAT), lambda i: (i, 0)),
            pl.BlockSpec((NC, NB, D_FEAT), lambda i: (0, i, 0)),
            pl.BlockSpec((NC, NB, D_FEAT), lambda i: (0, i, 0)),
            pl.BlockSpec((NB, D_FEAT), lambda i: (i, 0)),
            pl.BlockSpec((D_FEAT, 640), lambda i: (0, 0)),
        ],
        out_specs=[
            pl.BlockSpec((NB, D_FEAT), lambda i: (i, 0)),
            pl.BlockSpec((NB, 128), lambda i: (i, 0)),
            pl.BlockSpec((NB, 128), lambda i: (i, 0)),
            pl.BlockSpec((NB, D_FEAT), lambda i: (i, 0)),
        ],
        out_shape=[
            jax.ShapeDtypeStruct((N_NODES, D_FEAT), F32),
            jax.ShapeDtypeStruct((N_NODES, 128), jnp.int32),
            jax.ShapeDtypeStruct((N_NODES, 128), jnp.int32),
            jax.ShapeDtypeStruct((N_NODES, D_FEAT), F32),
        ],
    )(h, agga, aggb, nrin, wcat)


def _tc_act(u1, u2, ea_aug, we_aug):
    return pl.pallas_call(
        _act_body,
        grid=(E_HALF // EB,),
        in_specs=[
            pl.BlockSpec((EB, 128), lambda i: (i, 0)),
            pl.BlockSpec((EB, 128), lambda i: (i, 0)),
            pl.BlockSpec((EB, 17), lambda i: (i, 0)),
            pl.BlockSpec((17, 256), lambda i: (0, 0)),
        ],
        out_specs=pl.BlockSpec((EB, D_FEAT), lambda i: (i, 0)),
        out_shape=jax.ShapeDtypeStruct((E_HALF, D_FEAT), F32),
    )(u1, u2, ea_aug, we_aug)


def _tc_final(h, agga, aggb, nrin, wlin3, blin8, batch3):
    return pl.pallas_call(
        _final_body,
        grid=(NGRID,),
        in_specs=[
            pl.BlockSpec((NB, D_FEAT), lambda i: (i, 0)),
            pl.BlockSpec((NC, NB, D_FEAT), lambda i: (0, i, 0)),
            pl.BlockSpec((NC, NB, D_FEAT), lambda i: (0, i, 0)),
            pl.BlockSpec((NB, D_FEAT), lambda i: (i, 0)),
            pl.BlockSpec((D_FEAT, D_FEAT), lambda i: (0, 0)),
            pl.BlockSpec((8, D_FEAT), lambda i: (0, 0)),
            pl.BlockSpec((1, 1, NB), lambda i: (i, 0, 0)),
        ],
        out_specs=[
            pl.BlockSpec((NB, D_FEAT), lambda i: (i, 0)),
            pl.BlockSpec((N_GRAPHS, D_FEAT), lambda i: (0, 0)),
        ],
        out_shape=[
            jax.ShapeDtypeStruct((N_NODES, D_FEAT), F32),
            jax.ShapeDtypeStruct((N_GRAPHS, D_FEAT), F32),
        ],
    )(h, agga, aggb, nrin, wlin3, blin8, batch3)


E_HALF = N_EDGES // 2
EPW_H = E_HALF // NW
GCH = 40
GNCH = EPW_H // GCH
SCH = 40
SNCH = EPW_H // SCH


def _sc_gather(tdst, tsrc, dsti, srci):
    @functools.partial(
        pl.kernel,
        out_type=[jax.ShapeDtypeStruct((E_HALF, 128), jnp.int32),
                  jax.ShapeDtypeStruct((E_HALF, 128), jnp.int32)],
        mesh=_sc_mesh(),
        scratch_types=(
            [pltpu.VMEM((EPW_H,), jnp.int32)] * 2
            + [pltpu.VMEM((GCH, 128), jnp.int32)] * 8
            + [pltpu.SemaphoreType.DMA] * 16
        ),
    )
    def k(tdst_hbm, tsrc_hbm, dst_hbm, src_hbm, u1_hbm, u2_hbm,
          idxd, idxs, a0, a1, a2, a3, b0, b1, b2, b3,
          sa0, sa1, sa2, sa3, sb0, sb1, sb2, sb3,
          ta0, ta1, ta2, ta3, tb0, tb1, tb2, tb3):
        wid = lax.axis_index("s") * NC + lax.axis_index("c")
        base = wid * EPW_H
        pltpu.sync_copy(dst_hbm.at[pl.ds(base, EPW_H)], idxd)
        pltpu.sync_copy(src_hbm.at[pl.ds(base, EPW_H)], idxs)
        abuf = (a0, a1, a2, a3)
        bbuf = (b0, b1, b2, b3)
        sga = (sa0, sa1, sa2, sa3)
        sgb = (sb0, sb1, sb2, sb3)
        sta = (ta0, ta1, ta2, ta3)
        stb = (tb0, tb1, tb2, tb3)

        def fire(i, p):
            pltpu.async_copy(
                tdst_hbm.at[idxd.at[pl.ds(i * GCH, GCH)]], abuf[p], sga[p])
            pltpu.async_copy(
                tsrc_hbm.at[idxs.at[pl.ds(i * GCH, GCH)]], bbuf[p], sgb[p])

        def drain_store(q):
            pltpu.make_async_copy(
                abuf[q], u1_hbm.at[pl.ds(0, GCH)], sta[q]).wait()
            pltpu.make_async_copy(
                bbuf[q], u2_hbm.at[pl.ds(0, GCH)], stb[q]).wait()

        def visit(i, p, first=False, fire_next=True):
            q = (p + 2) % 4
            off = base + i * GCH
            pltpu.make_async_copy(
                tdst_hbm.at[idxd.at[pl.ds(0, GCH)]], abuf[p], sga[p]).wait()
            pltpu.make_async_copy(
                tsrc_hbm.at[idxs.at[pl.ds(0, GCH)]], bbuf[p], sgb[p]).wait()
            pltpu.async_copy(abuf[p], u1_hbm.at[pl.ds(off, GCH)], sta[p])
            pltpu.async_copy(bbuf[p], u2_hbm.at[pl.ds(off, GCH)], stb[p])
            if not first:
                drain_store(q)
            if fire_next:
                fire(i + 2, q)

        fire(0, 0)
        fire(1, 1)
        visit(0, 0, first=True)
        visit(1, 1, first=True)

        def body(g, cc):
            for t in range(4):
                visit(4 * g + 2 + t, (2 + t) % 4)
            return cc

        lax.fori_loop(0, (GNCH - 5) // 4, body, 0)
        visit(GNCH - 3, (GNCH - 3) % 4, fire_next=True)
        visit(GNCH - 2, (GNCH - 2) % 4, fire_next=False)
        visit(GNCH - 1, (GNCH - 1) % 4, fire_next=False)
        drain_store((GNCH - 2) % 4)
        drain_store((GNCH - 1) % 4)

    return k(tdst, tsrc, dsti, srci)


def _sc_scatter(msg, dsti):
    @functools.partial(
        pl.kernel,
        out_type=jax.ShapeDtypeStruct((NC, N_NODES, D_FEAT), F32),
        mesh=_sc_mesh(),
        scratch_types=[
            pltpu.VMEM((SCH,), jnp.int32),
            pltpu.VMEM((SCH,), jnp.int32),
            pltpu.VMEM((SCH,), jnp.int32),
            pltpu.VMEM((SCH,), jnp.int32),
            pltpu.VMEM((SCH, D_FEAT), F32),
            pltpu.VMEM((SCH, D_FEAT), F32),
            pltpu.VMEM((SCH, D_FEAT), F32),
            pltpu.VMEM((SCH, D_FEAT), F32),
            pltpu.VMEM((ZR, D_FEAT), F32),
            pltpu.VMEM_SHARED((N_NODES, D_FEAT), F32),
            pltpu.SemaphoreType.DMA,
            pltpu.SemaphoreType.DMA,
            pltpu.SemaphoreType.DMA,
            pltpu.SemaphoreType.DMA,
            pltpu.SemaphoreType.DMA,
            pltpu.SemaphoreType.DMA,
            pltpu.SemaphoreType.DMA,
            pltpu.SemaphoreType.DMA,
            pltpu.SemaphoreType.DMA,
            pltpu.SemaphoreType.DMA,
            pltpu.SemaphoreType.DMA,
            pltpu.SemaphoreType.DMA,
        ],
    )
    def k(msg_hbm, dst_hbm, out_hbm, i0, i1, i2, i3, m0, m1, m2, m3,
          zbuf, acc, si0, si1, si2, si3, sm0, sm1, sm2, sm3,
          sc0, sc1, sc2, sc3):
        c = lax.axis_index("c")
        s = lax.axis_index("s")
        wid = s * NC + c
        zero = jnp.zeros((16,), F32)
        ibufs = (i0, i1, i2, i3)
        mbufs = (m0, m1, m2, m3)
        sidx = (si0, si1, si2, si3)
        smsg = (sm0, sm1, sm2, sm3)
        ssc = (sc0, sc1, sc2, sc3)

        def zrow(r, cc):
            for j in range(D_FEAT // 16):
                zbuf[r, pl.ds(j * 16, 16)] = zero
            return cc

        lax.fori_loop(0, ZR, zrow, 0)

        @pl.when(s < NS - 1)
        def _():
            for t in range(SLAB // ZR):
                pltpu.async_copy(zbuf, acc.at[pl.ds(s * SLAB + t * ZR, ZR)],
                                 sc0)
            pltpu.async_copy(zbuf.at[pl.ds(0, SLAB % ZR)],
                             acc.at[pl.ds(s * SLAB + (SLAB // ZR) * ZR,
                                          SLAB % ZR)], sc1)
            for t in range(SLAB // ZR):
                pltpu.make_async_copy(
                    zbuf, acc.at[pl.ds(s * SLAB, ZR)], sc0).wait()
            pltpu.make_async_copy(
                zbuf.at[pl.ds(0, SLAB % ZR)],
                acc.at[pl.ds(s * SLAB, SLAB % ZR)], sc1).wait()

        @pl.when(s == NS - 1)
        def _():
            for t in range(LAST_SLAB // ZR):
                pltpu.async_copy(
                    zbuf, acc.at[pl.ds((NS - 1) * SLAB + t * ZR, ZR)], sc0)
            for t in range(LAST_SLAB // ZR):
                pltpu.make_async_copy(
                    zbuf, acc.at[pl.ds((NS - 1) * SLAB, ZR)], sc0).wait()

        plsc.subcore_barrier()

        def fire_load(i, p):
            off = wid * EPW_H + i * SCH
            pltpu.async_copy(dst_hbm.at[pl.ds(off, SCH)], ibufs[p], sidx[p])
            pltpu.async_copy(msg_hbm.at[pl.ds(off, SCH)], mbufs[p], smsg[p])

        fire_load(0, 0)
        fire_load(1, 1)

        def do_chunk(i, p):
            q = (p + 2) % 4
            pltpu.make_async_copy(
                dst_hbm.at[pl.ds(0, SCH)], ibufs[p], sidx[p]).wait()
            pltpu.make_async_copy(
                msg_hbm.at[pl.ds(0, SCH)], mbufs[p], smsg[p]).wait()
            pltpu.async_copy(mbufs[p], acc.at[ibufs[p]], ssc[p], add=True)

            @pl.when(i + 2 < SNCH)
            def _():
                @pl.when(i >= 2)
                def _():
                    pltpu.make_async_copy(
                        mbufs[q], acc.at[ibufs[q]], ssc[q]).wait()

                fire_load(i + 2, q)

        pltpu.make_async_copy(dst_hbm.at[pl.ds(0, SCH)], ibufs[0], sidx[0]).wait()
        pltpu.make_async_copy(msg_hbm.at[pl.ds(0, SCH)], mbufs[0], smsg[0]).wait()
        pltpu.async_copy(mbufs[0], acc.at[ibufs[0]], ssc[0], add=True)
        fire_load(2, 2)

        def body(g, cc):
            for t in range(4):
                do_chunk(4 * g + 1 + t, (1 + t) % 4)
            return cc

        lax.fori_loop(0, (SNCH - 1) // 4, body, 0)
        for p in range(4):
            pltpu.make_async_copy(
                mbufs[p], acc.at[ibufs[p]], ssc[p]).wait()
        plsc.subcore_barrier()

        @pl.when(s < NS - 1)
        def _():
            pltpu.sync_copy(acc.at[pl.ds(s * SLAB, SLAB)],
                            out_hbm.at[c, pl.ds(s * SLAB, SLAB)])

        @pl.when(s == NS - 1)
        def _():
            pltpu.sync_copy(acc.at[pl.ds((NS - 1) * SLAB, LAST_SLAB)],
                            out_hbm.at[c, pl.ds((NS - 1) * SLAB, LAST_SLAB)])

    return k(msg, dsti)


def kernel(x, edge_index, edge_attr, batch, Wf, bf, Ws, bs, Wlin, blin):
    src = edge_index[0]
    dst = edge_index[1]
    srch = (src[:E_HALF], src[E_HALF:])
    dsth = (dst[:E_HALF], dst[E_HALF:])
    ea_aug = jnp.concatenate(
        [edge_attr, jnp.ones((N_EDGES, 1), F32)], axis=1)
    eah = (ea_aug[:E_HALF], ea_aug[E_HALF:])
    batch3 = batch.reshape(NGRID, 1, NB)
    blin8 = jnp.broadcast_to(blin[None, :], (8, D_FEAT))

    h = x
    nr = None
    aggs = None
    for l in range(LAYERS):
        wcat = jnp.concatenate(
            [Wf[l][:128], Ws[l][:128], Wf[l][128:256], Ws[l][128:256],
             Wlin[128 * l:128 * (l + 1)]], axis=1)
        we_aug = jnp.concatenate(
            [jnp.concatenate([Wf[l][256:], Ws[l][256:]], axis=1),
             jnp.concatenate([bf[l], bs[l]])[None, :]], axis=0)
        if l == 0:
            tdst, tsrc, nr = _tc_proj0(h, wcat)
        else:
            h, tdst, tsrc, nr = _tc_proj_update(h, aggs[0], aggs[1], nr, wcat)
        u1a, u2a = _sc_gather(tdst, tsrc, dsth[0], srch[0])
        msg_a = _tc_act(u1a, u2a, eah[0], we_aug)
        u1b, u2b = _sc_gather(tdst, tsrc, dsth[1], srch[1])
        agg_a = _sc_scatter(msg_a, dsth[0])
        msg_b = _tc_act(u1b, u2b, eah[1], we_aug)
        agg_b = _sc_scatter(msg_b, dsth[1])
        aggs = (agg_a, agg_b)

    nr_out, out = _tc_final(h, aggs[0], aggs[1], nr, Wlin[384:], blin8, batch3)
    return (out, nr_out)


# speedup vs baseline: 4.1451x; 1.0158x over previous
import functools

import jax
import jax.numpy as jnp
from jax import lax
from jax.experimental import pallas as pl
from jax.experimental.pallas import tpu as pltpu
from jax.experimental.pallas import tpu_sc as plsc

F32 = jnp.float32
BF16 = jnp.bfloat16

N_NODES = 10000
N_EDGES = 320000
D_FEAT = 128
N_GRAPHS = 64
LAYERS = 3

NC = 2
NS = 16
NW = NC * NS
EPW = N_EDGES // NW
CH = 80
NCHUNK = EPW // CH
SLAB = 624
LAST_SLAB = N_NODES - (NS - 1) * SLAB
ZR = 64

@functools.cache
def _sc_mesh():
    return plsc.VectorSubcoreMesh(core_axis_name="c", subcore_axis_name="s",
                                  num_cores=NC, num_subcores=NS)

NB = 1000
NGRID = N_NODES // NB
EB = 6400


def _elu(u):
    return jnp.where(u > 0, u, jnp.exp(jnp.minimum(u, 0.0)) - 1.0)


def _pack2bf16(g, c):
    gu = lax.bitcast_convert_type(g.astype(BF16), jnp.uint16).astype(jnp.uint32)
    cu = lax.bitcast_convert_type(c.astype(BF16), jnp.uint16).astype(jnp.uint32)
    return lax.bitcast_convert_type(gu | (cu << 16), jnp.int32)


def _unpack2bf16(p):
    pu = lax.bitcast_convert_type(p, jnp.uint32)
    g = lax.bitcast_convert_type((pu & 0xFFFF).astype(jnp.uint16), BF16)
    c = lax.bitcast_convert_type((pu >> 16).astype(jnp.uint16), BF16)
    return g.astype(F32), c.astype(F32)


def _proj0_body(h_ref, w_ref, tdst_ref, tsrc_ref, nr_ref):
    t = jnp.dot(h_ref[...], w_ref[...], preferred_element_type=F32)
    tdst_ref[...] = _pack2bf16(t[:, :128], t[:, 128:256])
    tsrc_ref[...] = _pack2bf16(t[:, 256:384], t[:, 384:512])
    nr_ref[...] = t[:, 512:]


def _proj_update_body(h_ref, agga_ref, aggb_ref, nrin_ref, w_ref,
                      hout_ref, tdst_ref, tsrc_ref, nr_ref):
    h = _elu(h_ref[...] + agga_ref[0] + agga_ref[1]
             + aggb_ref[0] + aggb_ref[1])
    hout_ref[...] = h
    t = jnp.dot(h, w_ref[...], preferred_element_type=F32)
    tdst_ref[...] = _pack2bf16(t[:, :128], t[:, 128:256])
    tsrc_ref[...] = _pack2bf16(t[:, 256:384], t[:, 384:512])
    nr_ref[...] = nrin_ref[...] + t[:, 512:]


def _act_body(u1_ref, u2_ref, ea_ref, we_ref, msg_ref):
    e = jnp.dot(ea_ref[...], we_ref[...], preferred_element_type=F32)
    g1, c1 = _unpack2bf16(u1_ref[...])
    g2, c2 = _unpack2bf16(u2_ref[...])
    g = g1 + g2 + e[:, :128]
    s = c1 + c2 + e[:, 128:]
    gate = 1.0 / (1.0 + jnp.exp(-g))
    sp = jnp.maximum(s, 0.0) + jnp.log(1.0 + jnp.exp(-jnp.abs(s)))
    msg_ref[...] = gate * sp


def _final_body(h_ref, agga_ref, aggb_ref, nrin_ref, w_ref, b_ref,
                batch_ref, nr_ref, out_ref):
    i = pl.program_id(0)
    h = _elu(h_ref[...] + agga_ref[0] + agga_ref[1]
             + aggb_ref[0] + aggb_ref[1])
    nrb = (nrin_ref[...] + jnp.dot(h, w_ref[...], preferred_element_type=F32)
           + b_ref[0:1, :])
    nr_ref[...] = nrb
    bvec = batch_ref[0, 0, :]
    oh = (lax.broadcasted_iota(jnp.int32, (N_GRAPHS, NB), 0)
          == bvec[None, :]).astype(F32)

    @pl.when(i == 0)
    def _():
        out_ref[...] = jnp.zeros_like(out_ref)

    out_ref[...] += jnp.dot(oh, nrb, preferred_element_type=F32)


def _tc_proj0(h, wcat):
    return pl.pallas_call(
        _proj0_body,
        grid=(NGRID,),
        in_specs=[
            pl.BlockSpec((NB, D_FEAT), lambda i: (i, 0)),
            pl.BlockSpec((D_FEAT, 640), lambda i: (0, 0)),
        ],
        out_specs=[
            pl.BlockSpec((NB, 128), lambda i: (i, 0)),
            pl.BlockSpec((NB, 128), lambda i: (i, 0)),
            pl.BlockSpec((NB, D_FEAT), lambda i: (i, 0)),
        ],
        out_shape=[
            jax.ShapeDtypeStruct((N_NODES, 128), jnp.int32),
            jax.ShapeDtypeStruct((N_NODES, 128), jnp.int32),
            jax.ShapeDtypeStruct((N_NODES, D_FEAT), F32),
        ],
    )(h, wcat)


def _tc_proj_update(h, agga, aggb, nrin, wcat):
    return pl.pallas_call(
        _proj_update_body,
        grid=(NGRID,),
        in_specs=[
            pl.BlockSpec((NB, D_FEAT), lambda i: (i, 0)),
            pl.BlockSpec((NC, NB, D_FEAT), lambda i: (0, i, 0)),
            pl.BlockSpec((NC, NB, D_FEAT), lambda i: (0, i, 0)),
            pl.BlockSpec((NB, D_FEAT), lambda i: (i, 0)),
            pl.BlockSpec((D_FEAT, 640), lambda i: (0, 0)),
        ],
        out_specs=[
            pl.BlockSpec((NB, D_FEAT), lambda i: (i, 0)),
            pl.BlockSpec((NB, 128), lambda i: (i, 0)),
            pl.BlockSpec((NB, 128), lambda i: (i, 0)),
            pl.BlockSpec((NB, D_FEAT), lambda i: (i, 0)),
        ],
        out_shape=[
            jax.ShapeDtypeStruct((N_NODES, D_FEAT), F32),
            jax.ShapeDtypeStruct((N_NODES, 128), jnp.int32),
            jax.ShapeDtypeStruct((N_NODES, 128), jnp.int32),
            jax.ShapeDtypeStruct((N_NODES, D_FEAT), F32),
        ],
    )(h, agga, aggb, nrin, wcat)


def _tc_act(u1, u2, ea_aug, we_aug):
    return pl.pallas_call(
        _act_body,
        grid=(E_HALF // EB,),
        in_specs=[
            pl.BlockSpec((EB, 128), lambda i: (i, 0)),
            pl.BlockSpec((EB, 128), lambda i: (i, 0)),
            pl.BlockSpec((EB, 17), lambda i: (i, 0)),
            pl.BlockSpec((17, 256), lambda i: (0, 0)),
        ],
        out_specs=pl.BlockSpec((EB, D_FEAT), lambda i: (i, 0)),
        out_shape=jax.ShapeDtypeStruct((E_HALF, D_FEAT), F32),
    )(u1, u2, ea_aug, we_aug)


def _tc_final(h, agga, aggb, nrin, wlin3, blin8, batch3):
    return pl.pallas_call(
        _final_body,
        grid=(NGRID,),
        in_specs=[
            pl.BlockSpec((NB, D_FEAT), lambda i: (i, 0)),
            pl.BlockSpec((NC, NB, D_FEAT), lambda i: (0, i, 0)),
            pl.BlockSpec((NC, NB, D_FEAT), lambda i: (0, i, 0)),
            pl.BlockSpec((NB, D_FEAT), lambda i: (i, 0)),
            pl.BlockSpec((D_FEAT, D_FEAT), lambda i: (0, 0)),
            pl.BlockSpec((8, D_FEAT), lambda i: (0, 0)),
            pl.BlockSpec((1, 1, NB), lambda i: (i, 0, 0)),
        ],
        out_specs=[
            pl.BlockSpec((NB, D_FEAT), lambda i: (i, 0)),
            pl.BlockSpec((N_GRAPHS, D_FEAT), lambda i: (0, 0)),
        ],
        out_shape=[
            jax.ShapeDtypeStruct((N_NODES, D_FEAT), F32),
            jax.ShapeDtypeStruct((N_GRAPHS, D_FEAT), F32),
        ],
    )(h, agga, aggb, nrin, wlin3, blin8, batch3)


E_HALF = N_EDGES // 2
EPW_H = E_HALF // NW
GCH = 40
GNCH = EPW_H // GCH
SCH = 40
SNCH = EPW_H // SCH


def _sc_gather(tdst, tsrc, dsti, srci):
    @functools.partial(
        pl.kernel,
        out_type=[jax.ShapeDtypeStruct((E_HALF, 128), jnp.int32),
                  jax.ShapeDtypeStruct((E_HALF, 128), jnp.int32)],
        mesh=_sc_mesh(),
        scratch_types=(
            [pltpu.VMEM((EPW_H,), jnp.int32)] * 2
            + [pltpu.VMEM((GCH, 128), jnp.int32)] * 8
            + [pltpu.SemaphoreType.DMA] * 16
        ),
    )
    def k(tdst_hbm, tsrc_hbm, dst_hbm, src_hbm, u1_hbm, u2_hbm,
          idxd, idxs, a0, a1, a2, a3, b0, b1, b2, b3,
          sa0, sa1, sa2, sa3, sb0, sb1, sb2, sb3,
          ta0, ta1, ta2, ta3, tb0, tb1, tb2, tb3):
        wid = lax.axis_index("s") * NC + lax.axis_index("c")
        base = wid * EPW_H
        cpd = pltpu.async_copy(dst_hbm.at[pl.ds(base, EPW_H)], idxd, ta0)
        cps = pltpu.async_copy(src_hbm.at[pl.ds(base, EPW_H)], idxs, tb0)
        cpd.wait()
        cps.wait()
        abuf = (a0, a1, a2, a3)
        bbuf = (b0, b1, b2, b3)
        sga = (sa0, sa1, sa2, sa3)
        sgb = (sb0, sb1, sb2, sb3)
        sta = (ta0, ta1, ta2, ta3)
        stb = (tb0, tb1, tb2, tb3)

        def fire(i, p):
            pltpu.async_copy(
                tdst_hbm.at[idxd.at[pl.ds(i * GCH, GCH)]], abuf[p], sga[p])
            pltpu.async_copy(
                tsrc_hbm.at[idxs.at[pl.ds(i * GCH, GCH)]], bbuf[p], sgb[p])

        def drain_store(q):
            pltpu.make_async_copy(
                abuf[q], u1_hbm.at[pl.ds(0, GCH)], sta[q]).wait()
            pltpu.make_async_copy(
                bbuf[q], u2_hbm.at[pl.ds(0, GCH)], stb[q]).wait()

        def visit(i, p, first=False, fire_next=True):
            q = (p + 2) % 4
            off = base + i * GCH
            pltpu.make_async_copy(
                tdst_hbm.at[idxd.at[pl.ds(0, GCH)]], abuf[p], sga[p]).wait()
            pltpu.make_async_copy(
                tsrc_hbm.at[idxs.at[pl.ds(0, GCH)]], bbuf[p], sgb[p]).wait()
            pltpu.async_copy(abuf[p], u1_hbm.at[pl.ds(off, GCH)], sta[p])
            pltpu.async_copy(bbuf[p], u2_hbm.at[pl.ds(off, GCH)], stb[p])
            if not first:
                drain_store(q)
            if fire_next:
                fire(i + 2, q)

        fire(0, 0)
        fire(1, 1)
        visit(0, 0, first=True)
        visit(1, 1, first=True)

        def body(g, cc):
            for t in range(4):
                visit(4 * g + 2 + t, (2 + t) % 4)
            return cc

        lax.fori_loop(0, (GNCH - 5) // 4, body, 0)
        visit(GNCH - 3, (GNCH - 3) % 4, fire_next=True)
        visit(GNCH - 2, (GNCH - 2) % 4, fire_next=False)
        visit(GNCH - 1, (GNCH - 1) % 4, fire_next=False)
        drain_store((GNCH - 2) % 4)
        drain_store((GNCH - 1) % 4)

    return k(tdst, tsrc, dsti, srci)


def _sc_scatter(msg, dsti):
    @functools.partial(
        pl.kernel,
        out_type=jax.ShapeDtypeStruct((NC, N_NODES, D_FEAT), F32),
        mesh=_sc_mesh(),
        scratch_types=[
            pltpu.VMEM((SCH,), jnp.int32),
            pltpu.VMEM((SCH,), jnp.int32),
            pltpu.VMEM((SCH,), jnp.int32),
            pltpu.VMEM((SCH,), jnp.int32),
            pltpu.VMEM((SCH, D_FEAT), F32),
            pltpu.VMEM((SCH, D_FEAT), F32),
            pltpu.VMEM((SCH, D_FEAT), F32),
            pltpu.VMEM((SCH, D_FEAT), F32),
            pltpu.VMEM((ZR, D_FEAT), F32),
            pltpu.VMEM_SHARED((N_NODES, D_FEAT), F32),
            pltpu.SemaphoreType.DMA,
            pltpu.SemaphoreType.DMA,
            pltpu.SemaphoreType.DMA,
            pltpu.SemaphoreType.DMA,
            pltpu.SemaphoreType.DMA,
            pltpu.SemaphoreType.DMA,
            pltpu.SemaphoreType.DMA,
            pltpu.SemaphoreType.DMA,
            pltpu.SemaphoreType.DMA,
            pltpu.SemaphoreType.DMA,
            pltpu.SemaphoreType.DMA,
            pltpu.SemaphoreType.DMA,
        ],
    )
    def k(msg_hbm, dst_hbm, out_hbm, i0, i1, i2, i3, m0, m1, m2, m3,
          zbuf, acc, si0, si1, si2, si3, sm0, sm1, sm2, sm3,
          sc0, sc1, sc2, sc3):
        c = lax.axis_index("c")
        s = lax.axis_index("s")
        wid = s * NC + c
        zero = jnp.zeros((16,), F32)
        ibufs = (i0, i1, i2, i3)
        fired_early = True
        mbufs = (m0, m1, m2, m3)
        sidx = (si0, si1, si2, si3)
        smsg = (sm0, sm1, sm2, sm3)
        ssc = (sc0, sc1, sc2, sc3)

        def fire_load(i, p):
            off = wid * EPW_H + i * SCH
            pltpu.async_copy(dst_hbm.at[pl.ds(off, SCH)], ibufs[p], sidx[p])
            pltpu.async_copy(msg_hbm.at[pl.ds(off, SCH)], mbufs[p], smsg[p])

        fire_load(0, 0)
        fire_load(1, 1)

        def zrow(r, cc):
            for j in range(D_FEAT // 16):
                zbuf[r, pl.ds(j * 16, 16)] = zero
            return cc

        lax.fori_loop(0, ZR, zrow, 0)

        @pl.when(s < NS - 1)
        def _():
            for t in range(SLAB // ZR):
                pltpu.async_copy(zbuf, acc.at[pl.ds(s * SLAB + t * ZR, ZR)],
                                 sc0)
            pltpu.async_copy(zbuf.at[pl.ds(0, SLAB % ZR)],
                             acc.at[pl.ds(s * SLAB + (SLAB // ZR) * ZR,
                                          SLAB % ZR)], sc1)
            for t in range(SLAB // ZR):
                pltpu.make_async_copy(
                    zbuf, acc.at[pl.ds(s * SLAB, ZR)], sc0).wait()
            pltpu.make_async_copy(
                zbuf.at[pl.ds(0, SLAB % ZR)],
                acc.at[pl.ds(s * SLAB, SLAB % ZR)], sc1).wait()

        @pl.when(s == NS - 1)
        def _():
            for t in range(LAST_SLAB // ZR):
                pltpu.async_copy(
                    zbuf, acc.at[pl.ds((NS - 1) * SLAB + t * ZR, ZR)], sc0)
            for t in range(LAST_SLAB // ZR):
                pltpu.make_async_copy(
                    zbuf, acc.at[pl.ds((NS - 1) * SLAB, ZR)], sc0).wait()

        plsc.subcore_barrier()

        def do_chunk(i, p):
            q = (p + 2) % 4
            pltpu.make_async_copy(
                dst_hbm.at[pl.ds(0, SCH)], ibufs[p], sidx[p]).wait()
            pltpu.make_async_copy(
                msg_hbm.at[pl.ds(0, SCH)], mbufs[p], smsg[p]).wait()
            pltpu.async_copy(mbufs[p], acc.at[ibufs[p]], ssc[p], add=True)

            @pl.when(i + 2 < SNCH)
            def _():
                @pl.when(i >= 2)
                def _():
                    pltpu.make_async_copy(
                        mbufs[q], acc.at[ibufs[q]], ssc[q]).wait()

                fire_load(i + 2, q)

        pltpu.make_async_copy(dst_hbm.at[pl.ds(0, SCH)], ibufs[0], sidx[0]).wait()
        pltpu.make_async_copy(msg_hbm.at[pl.ds(0, SCH)], mbufs[0], smsg[0]).wait()
        pltpu.async_copy(mbufs[0], acc.at[ibufs[0]], ssc[0], add=True)
        fire_load(2, 2)

        def body(g, cc):
            for t in range(4):
                do_chunk(4 * g + 1 + t, (1 + t) % 4)
            return cc

        lax.fori_loop(0, (SNCH - 1) // 4, body, 0)
        for p in range(4):
            pltpu.make_async_copy(
                mbufs[p], acc.at[ibufs[p]], ssc[p]).wait()
        plsc.subcore_barrier()

        @pl.when(s < NS - 1)
        def _():
            pltpu.sync_copy(acc.at[pl.ds(s * SLAB, SLAB)],
                            out_hbm.at[c, pl.ds(s * SLAB, SLAB)])

        @pl.when(s == NS - 1)
        def _():
            pltpu.sync_copy(acc.at[pl.ds((NS - 1) * SLAB, LAST_SLAB)],
                            out_hbm.at[c, pl.ds((NS - 1) * SLAB, LAST_SLAB)])

    return k(msg, dsti)


def kernel(x, edge_index, edge_attr, batch, Wf, bf, Ws, bs, Wlin, blin):
    src = edge_index[0]
    dst = edge_index[1]
    srch = (src[:E_HALF], src[E_HALF:])
    dsth = (dst[:E_HALF], dst[E_HALF:])
    ea_aug = jnp.concatenate(
        [edge_attr, jnp.ones((N_EDGES, 1), F32)], axis=1)
    eah = (ea_aug[:E_HALF], ea_aug[E_HALF:])
    batch3 = batch.reshape(NGRID, 1, NB)
    blin8 = jnp.broadcast_to(blin[None, :], (8, D_FEAT))

    h = x
    nr = None
    aggs = None
    for l in range(LAYERS):
        wcat = jnp.concatenate(
            [Wf[l][:128], Ws[l][:128], Wf[l][128:256], Ws[l][128:256],
             Wlin[128 * l:128 * (l + 1)]], axis=1)
        we_aug = jnp.concatenate(
            [jnp.concatenate([Wf[l][256:], Ws[l][256:]], axis=1),
             jnp.concatenate([bf[l], bs[l]])[None, :]], axis=0)
        if l == 0:
            tdst, tsrc, nr = _tc_proj0(h, wcat)
        else:
            h, tdst, tsrc, nr = _tc_proj_update(h, aggs[0], aggs[1], nr, wcat)
        u1a, u2a = _sc_gather(tdst, tsrc, dsth[0], srch[0])
        msg_a = _tc_act(u1a, u2a, eah[0], we_aug)
        u1b, u2b = _sc_gather(tdst, tsrc, dsth[1], srch[1])
        agg_a = _sc_scatter(msg_a, dsth[0])
        msg_b = _tc_act(u1b, u2b, eah[1], we_aug)
        agg_b = _sc_scatter(msg_b, dsth[1])
        aggs = (agg_a, agg_b)

    nr_out, out = _tc_final(h, aggs[0], aggs[1], nr, Wlin[384:], blin8, batch3)
    return (out, nr_out)
